# baseline (speedup 1.0000x reference)
import functools

import jax
import jax.numpy as jnp
from jax import lax
from jax.experimental import pallas as pl
from jax.experimental.pallas import tpu as pltpu
from jax.experimental.pallas import tpu_sc as plsc

N = 10000
E = 320000
D = 128
H = 64
NCLS = 6

NP = 10240
NC = 2
NS = 16
NW = NC * NS
CH = 128
NCHK = 80
TPW = NCHK * CH
ET = NW * TPW

NEG = -1e30

_SC_MESH = plsc.VectorSubcoreMesh(core_axis_name="c", subcore_axis_name="s")


def _make_sc_edge(dc, NB, LA):
    @functools.partial(
        pl.kernel,
        mesh=_SC_MESH,
        compiler_params=pltpu.CompilerParams(
            needs_layout_passes=False, use_tc_tiling_on_sc=False,
            disable_bounds_checks=True),
        out_type=[
            jax.ShapeDtypeStruct((NC, NP, dc), jnp.float32),
            jax.ShapeDtypeStruct((NC, NP), jnp.float32),
        ],
        scratch_types=(
            [
                pltpu.VMEM((NCHK, CH), jnp.int32),
                pltpu.VMEM((NCHK, CH), jnp.int32),
                pltpu.VMEM((NP,), jnp.float32),
                pltpu.VMEM((NP,), jnp.float32),
            ]
            + [pltpu.VMEM((CH + 16,), jnp.float32) for _ in range(NB)]
            + [pltpu.VMEM((CH, dc), jnp.float32) for _ in range(NB)]
            + [
                pltpu.VMEM_SHARED((NP, dc), jnp.float32),
                pltpu.VMEM_SHARED((NP,), jnp.float32),
            ]
            + [pltpu.SemaphoreType.DMA for _ in range(3 * NB)]
        ),
    )
    def sc_edge(ei_hbm, asrc_hbm, adst_hbm, h_hbm, zrow_hbm,
                zden_hbm, rawout_hbm, denout_hbm,
                src_v, dst_v, as_v, ad_v, *rest):
        exs = rest[0:NB]
        rows = rest[NB:2 * NB]
        raw_sh, den_sh = rest[2 * NB], rest[2 * NB + 1]
        gsem = rest[2 * NB + 2:2 * NB + 2 + NB]
        rsem = rest[2 * NB + 2 + NB:2 * NB + 2 + 2 * NB]
        dsem = rest[2 * NB + 2 + 2 * NB:2 * NB + 2 + 3 * NB]

        c = lax.axis_index("c")
        s = lax.axis_index("s")
        wid = c * NS + s

        pltpu.sync_copy(ei_hbm.at[0, wid], src_v)
        pltpu.sync_copy(ei_hbm.at[1, wid], dst_v)
        pltpu.sync_copy(asrc_hbm, as_v)
        pltpu.sync_copy(adst_hbm, ad_v)

        @pl.when(s == 0)
        def _init():
            pltpu.sync_copy(zrow_hbm, raw_sh)
            pltpu.sync_copy(zden_hbm, den_sh)

        plsc.subcore_barrier()

        for p in range(LA):
            pltpu.async_copy(h_hbm.at[src_v.at[p]], rows[p], gsem[p])

        def quad_body(t, carry):
            for b in range(NB):
                j = t * NB + b
                b2 = (b + LA) % NB

                @pl.when(j >= NB - LA)
                def _drain():
                    jd = j + LA - NB
                    pltpu.make_async_copy(
                        rows[b2], raw_sh.at[dst_v.at[jd]], rsem[b2]).wait()
                    pltpu.make_async_copy(
                        exs[b2].at[pl.ds(0, CH)], den_sh.at[dst_v.at[jd]],
                        dsem[b2]).wait()

                @pl.when(j + LA < NCHK)
                def _fire():
                    pltpu.async_copy(
                        h_hbm.at[src_v.at[j + LA]], rows[b2], gsem[b2])

                for k in range(CH // 16):
                    sidx = src_v[j, pl.ds(k * 16, 16)]
                    didx = dst_v[j, pl.ds(k * 16, 16)]
                    al = (plsc.load_gather(as_v, [sidx])
                          + plsc.load_gather(ad_v, [didx]))
                    al = jnp.maximum(al, 0.2 * al)
                    exs[b][pl.ds(k * 16, 16)] = jnp.exp(al)

                pltpu.make_async_copy(
                    h_hbm.at[src_v.at[j]], rows[b], gsem[b]).wait()

                def row_body(i, carry2):
                    e0 = i * 16
                    exv = exs[b][pl.ds(e0, 16)]
                    for r in range(16):
                        ev = exv[r]
                        for g in range(dc // 16):
                            rows[b][e0 + r, pl.ds(g * 16, 16)] = (
                                rows[b][e0 + r, pl.ds(g * 16, 16)] * ev)
                    return carry2

                lax.fori_loop(0, CH // 16, row_body, 0)

                pltpu.async_copy(rows[b], raw_sh.at[dst_v.at[j]], rsem[b],
                                 add=True)
                pltpu.async_copy(exs[b].at[pl.ds(0, CH)],
                                 den_sh.at[dst_v.at[j]], dsem[b], add=True)
            return carry

        lax.fori_loop(0, NCHK // NB, quad_body, 0)

        for j in range(NCHK - (NB - LA), NCHK):
            b = j % NB
            pltpu.make_async_copy(rows[b], raw_sh.at[dst_v.at[j]],
                                  rsem[b]).wait()
            pltpu.make_async_copy(exs[b].at[pl.ds(0, CH)],
                                  den_sh.at[dst_v.at[j]], dsem[b]).wait()

        plsc.subcore_barrier()

        rps = NP // NS
        pltpu.sync_copy(raw_sh.at[pl.ds(s * rps, rps)],
                        rawout_hbm.at[c, pl.ds(s * rps, rps)])
        pltpu.sync_copy(den_sh.at[pl.ds(s * rps, rps)],
                        denout_hbm.at[c, pl.ds(s * rps, rps)])

    return sc_edge


_sc_edge64 = _make_sc_edge(H, 5, 3)
_sc_edge16 = _make_sc_edge(16, 8, 4)

_RB = 2048
_GRID = NP // _RB


def _tc1_body(x_ref, w_ref, asa_ref, ada_ref, h_ref, aso_ref, ado_ref):
    h = jnp.dot(x_ref[...], w_ref[...], preferred_element_type=jnp.float32)
    h_ref[...] = h
    rows = (pl.program_id(0) * _RB
            + lax.broadcasted_iota(jnp.int32, (_RB, 1), 0))
    valid = rows < N
    aso = jnp.sum(h * asa_ref[...], axis=1, keepdims=True)
    ado = jnp.sum(h * ada_ref[...], axis=1, keepdims=True)
    aso_ref[...] = jnp.where(valid, aso, NEG)
    ado_ref[...] = jnp.where(valid, ado, NEG)


def _tc2_body(raw_ref, den_ref, h1_ref, as1_ref, ad1_ref, b1_ref, w2_ref,
              as2_ref, ad2_ref, h2_ref, aso_ref, ado_ref):
    al = as1_ref[...] + ad1_ref[...]
    exs = jnp.exp(jnp.maximum(al, 0.2 * al))
    r = raw_ref[0] + raw_ref[1] + exs * h1_ref[...]
    d = den_ref[0] + den_ref[1] + exs
    o1 = r / (d + 1e-16) + b1_ref[...]
    hin = jnp.where(o1 > 0, o1, jnp.exp(jnp.minimum(o1, 0.0)) - 1.0)
    h2 = jnp.dot(hin, w2_ref[...], preferred_element_type=jnp.float32)
    h2_ref[...] = h2
    rows = (pl.program_id(0) * _RB
            + lax.broadcasted_iota(jnp.int32, (_RB, 1), 0))
    valid = rows < N
    aso = jnp.sum(h2 * as2_ref[...], axis=1, keepdims=True)
    ado = jnp.sum(h2 * ad2_ref[...], axis=1, keepdims=True)
    aso_ref[...] = jnp.where(valid, aso, NEG)
    ado_ref[...] = jnp.where(valid, ado, NEG)


def _tc3_body(raw_ref, den_ref, h2_ref, as2_ref, ad2_ref, b2_ref, out_ref):
    al = as2_ref[...] + ad2_ref[...]
    exs = jnp.exp(jnp.maximum(al, 0.2 * al))
    r = raw_ref[0] + raw_ref[1] + exs * h2_ref[...]
    d = den_ref[0] + den_ref[1] + exs
    out_ref[...] = r / (d + 1e-16) + b2_ref[...]


def kernel(x, edge_index, W1, att_src1, att_dst1, b1, W2, att_src2, att_dst2, b2):
    f32 = jnp.float32

    xp = jnp.pad(x, ((0, NP - N), (0, 0)))
    ei3 = jnp.pad(edge_index, ((0, 0), (0, ET - E)),
                  constant_values=N).reshape(2, NW, NCHK, CH)

    zrow64 = jnp.zeros((NP, H), f32)
    zrow16 = jnp.zeros((NP, 16), f32)
    zden = jnp.zeros((NP,), f32)

    w2p = jnp.pad(W2, ((0, 0), (0, 16 - NCLS)))
    as2p = jnp.pad(att_src2, ((0, 0), (0, 16 - NCLS)))
    ad2p = jnp.pad(att_dst2, ((0, 0), (0, 16 - NCLS)))
    b2p = jnp.pad(b2, (0, 16 - NCLS)).reshape(1, 16)

    h1, as1, ad1 = pl.pallas_call(
        _tc1_body,
        grid=(_GRID,),
        in_specs=[
            pl.BlockSpec((_RB, D), lambda i: (i, 0)),
            pl.BlockSpec((D, H), lambda i: (0, 0)),
            pl.BlockSpec((1, H), lambda i: (0, 0)),
            pl.BlockSpec((1, H), lambda i: (0, 0)),
        ],
        out_specs=[
            pl.BlockSpec((_RB, H), lambda i: (i, 0)),
            pl.BlockSpec((_RB, 1), lambda i: (i, 0)),
            pl.BlockSpec((_RB, 1), lambda i: (i, 0)),
        ],
        out_shape=[
            jax.ShapeDtypeStruct((NP, H), f32),
            jax.ShapeDtypeStruct((NP, 1), f32),
            jax.ShapeDtypeStruct((NP, 1), f32),
        ],
    )(xp, W1, att_src1, att_dst1)

    raw1, den1 = _sc_edge64(ei3, as1.reshape(NP), ad1.reshape(NP),
                            h1, zrow64, zden)

    h2, as2, ad2 = pl.pallas_call(
        _tc2_body,
        grid=(_GRID,),
        in_specs=[
            pl.BlockSpec((NC, _RB, H), lambda i: (0, i, 0)),
            pl.BlockSpec((NC, _RB, 1), lambda i: (0, i, 0)),
            pl.BlockSpec((_RB, H), lambda i: (i, 0)),
            pl.BlockSpec((_RB, 1), lambda i: (i, 0)),
            pl.BlockSpec((_RB, 1), lambda i: (i, 0)),
            pl.BlockSpec((1, H), lambda i: (0, 0)),
            pl.BlockSpec((H, 16), lambda i: (0, 0)),
            pl.BlockSpec((1, 16), lambda i: (0, 0)),
            pl.BlockSpec((1, 16), lambda i: (0, 0)),
        ],
        out_specs=[
            pl.BlockSpec((_RB, 16), lambda i: (i, 0)),
            pl.BlockSpec((_RB, 1), lambda i: (i, 0)),
            pl.BlockSpec((_RB, 1), lambda i: (i, 0)),
        ],
        out_shape=[
            jax.ShapeDtypeStruct((NP, 16), f32),
            jax.ShapeDtypeStruct((NP, 1), f32),
            jax.ShapeDtypeStruct((NP, 1), f32),
        ],
    )(raw1, den1.reshape(NC, NP, 1), h1, as1, ad1, b1.reshape(1, H), w2p,
      as2p, ad2p)

    raw2, den2 = _sc_edge16(ei3, as2.reshape(NP), ad2.reshape(NP),
                            h2, zrow16, zden)

    out = pl.pallas_call(
        _tc3_body,
        grid=(_GRID,),
        in_specs=[
            pl.BlockSpec((NC, _RB, 16), lambda i: (0, i, 0)),
            pl.BlockSpec((NC, _RB, 1), lambda i: (0, i, 0)),
            pl.BlockSpec((_RB, 16), lambda i: (i, 0)),
            pl.BlockSpec((_RB, 1), lambda i: (i, 0)),
            pl.BlockSpec((_RB, 1), lambda i: (i, 0)),
            pl.BlockSpec((1, 16), lambda i: (0, 0)),
        ],
        out_specs=pl.BlockSpec((_RB, 16), lambda i: (i, 0)),
        out_shape=jax.ShapeDtypeStruct((NP, 16), f32),
    )(raw2, den2.reshape(NC, NP, 1), h2, as2, ad2, b2p)

    return out[:N, :NCLS]


# speedup vs baseline: 1.0977x; 1.0977x over previous
import functools

import jax
import jax.numpy as jnp
import numpy as np
from jax import lax
from jax.experimental import pallas as pl
from jax.experimental.pallas import tpu as pltpu
from jax.experimental.pallas import tpu_sc as plsc

N = 10000
E = 320000
D = 128
H = 64
NCLS = 6

NP = 10240
NC = 2
NS = 16
NW = NC * NS
CH = 128
NCHK = 80
TPW = NCHK * CH
ET = NW * TPW

NEG = -1e30

_SC_MESH = plsc.VectorSubcoreMesh(core_axis_name="c", subcore_axis_name="s")


def _make_sc_edge(dc, NB, LA, bf, FB=2):
    row_dt = jnp.bfloat16 if bf else jnp.float32
    nf = FB if bf else 0

    @functools.partial(
        pl.kernel,
        mesh=_SC_MESH,
        compiler_params=pltpu.CompilerParams(
            needs_layout_passes=False, use_tc_tiling_on_sc=False,
            disable_bounds_checks=True),
        out_type=[
            jax.ShapeDtypeStruct((NC, NP, dc), jnp.float32),
            jax.ShapeDtypeStruct((NC, NP), jnp.float32),
        ],
        scratch_types=(
            [
                pltpu.VMEM((NCHK, CH), jnp.int32),
                pltpu.VMEM((NCHK, CH), jnp.int32),
                pltpu.VMEM((NP,), jnp.float32),
                pltpu.VMEM((NP,), jnp.float32),
            ]
            + [pltpu.VMEM((CH + 16,), jnp.float32) for _ in range(NB)]
            + [pltpu.VMEM((CH, dc), row_dt) for _ in range(NB)]
            + [pltpu.VMEM((CH, dc), jnp.float32) for _ in range(nf)]
            + [
                pltpu.VMEM_SHARED((NP, dc), jnp.float32),
                pltpu.VMEM_SHARED((NP,), jnp.float32),
            ]
            + [pltpu.SemaphoreType.DMA
               for _ in range(2 * NB + (FB if bf else NB))]
        ),
    )
    def sc_edge(ei_hbm, asrc_hbm, adst_hbm, h_hbm, zrow_hbm,
                zden_hbm, rawout_hbm, denout_hbm,
                src_v, dst_v, as_v, ad_v, *rest):
        exs = rest[0:NB]
        rows = rest[NB:2 * NB]
        frows = rest[2 * NB:2 * NB + nf] if bf else rows
        o = 2 * NB + nf
        raw_sh, den_sh = rest[o], rest[o + 1]
        sems = rest[o + 2:]
        gsem = sems[0:NB]
        dsem = sems[NB:2 * NB]
        rsem = sems[2 * NB:]

        c = lax.axis_index("c")
        s = lax.axis_index("s")
        wid = c * NS + s

        pltpu.sync_copy(ei_hbm.at[0, wid], src_v)
        pltpu.sync_copy(ei_hbm.at[1, wid], dst_v)
        pltpu.sync_copy(asrc_hbm, as_v)
        pltpu.sync_copy(adst_hbm, ad_v)

        @pl.when(s == 0)
        def _init():
            pltpu.sync_copy(zrow_hbm, raw_sh)
            pltpu.sync_copy(zden_hbm, den_sh)

        plsc.subcore_barrier()

        for p in range(LA):
            pltpu.async_copy(h_hbm.at[src_v.at[p]], rows[p], gsem[p])

        def quad_body(t, carry):
            for b in range(NB):
                j = t * NB + b
                b2 = (b + LA) % NB

                @pl.when(j >= NB - LA)
                def _drain_den():
                    jd = j + LA - NB
                    pltpu.make_async_copy(
                        exs[b2].at[pl.ds(0, CH)], den_sh.at[dst_v.at[jd]],
                        dsem[b2]).wait()

                if not bf:
                    @pl.when(j >= NB - LA)
                    def _drain_rows():
                        jd = j + LA - NB
                        pltpu.make_async_copy(
                            rows[b2], raw_sh.at[dst_v.at[jd]],
                            rsem[b2]).wait()

                @pl.when(j + LA < NCHK)
                def _fire():
                    pltpu.async_copy(
                        h_hbm.at[src_v.at[j + LA]], rows[b2], gsem[b2])

                for k in range(CH // 16):
                    sidx = src_v[j, pl.ds(k * 16, 16)]
                    didx = dst_v[j, pl.ds(k * 16, 16)]
                    al = (plsc.load_gather(as_v, [sidx])
                          + plsc.load_gather(ad_v, [didx]))
                    al = jnp.maximum(al, 0.2 * al)
                    exs[b][pl.ds(k * 16, 16)] = jnp.exp(al)

                pltpu.make_async_copy(
                    h_hbm.at[src_v.at[j]], rows[b], gsem[b]).wait()

                if bf:
                    f = b % FB

                    @pl.when(j >= FB)
                    def _drain_rows_bf():
                        pltpu.make_async_copy(
                            frows[f], raw_sh.at[dst_v.at[j - FB]],
                            rsem[f]).wait()


                    def row_body(i, carry2):
                        e0 = i * 16
                        exv = exs[b][pl.ds(e0, 16)]
                        for r in range(16):
                            ev = exv[r]
                            e = e0 + r
                            for g in range(dc // 32):
                                bfv = rows[b][e, pl.ds(g * 32, 32)]
                                lo, hi = plsc.unpack(
                                    bfv, format=plsc.PackFormat.INTERLEAVED)
                                frows[f][e, pl.ds(g * 32, 16)] = lo * ev
                                frows[f][e, pl.ds(g * 32 + 16, 16)] = hi * ev
                        return carry2
                else:
                    f = b

                    def row_body(i, carry2):
                        e0 = i * 16
                        exv = exs[b][pl.ds(e0, 16)]
                        for r in range(16):
                            ev = exv[r]
                            for g in range(dc // 16):
                                rows[b][e0 + r, pl.ds(g * 16, 16)] = (
                                    rows[b][e0 + r, pl.ds(g * 16, 16)] * ev)
                        return carry2

                lax.fori_loop(0, CH // 16, row_body, 0)

                pltpu.async_copy(frows[f], raw_sh.at[dst_v.at[j]], rsem[f],
                                 add=True)
                pltpu.async_copy(exs[b].at[pl.ds(0, CH)],
                                 den_sh.at[dst_v.at[j]], dsem[b], add=True)
            return carry

        lax.fori_loop(0, NCHK // NB, quad_body, 0)

        nr = FB if bf else NB - LA
        for j in range(NCHK - nr, NCHK):
            f = (j % NB) % FB if bf else j % NB
            pltpu.make_async_copy(frows[f], raw_sh.at[dst_v.at[j]],
                                  rsem[f]).wait()
        for j in range(NCHK - (NB - LA), NCHK):
            b = j % NB
            pltpu.make_async_copy(exs[b].at[pl.ds(0, CH)],
                                  den_sh.at[dst_v.at[j]], dsem[b]).wait()

        plsc.subcore_barrier()

        rps = NP // NS
        pltpu.sync_copy(raw_sh.at[pl.ds(s * rps, rps)],
                        rawout_hbm.at[c, pl.ds(s * rps, rps)])
        pltpu.sync_copy(den_sh.at[pl.ds(s * rps, rps)],
                        denout_hbm.at[c, pl.ds(s * rps, rps)])

    return sc_edge


_sc_edge64 = _make_sc_edge(H, 4, 2, True)
_sc_edge16 = _make_sc_edge(16, 8, 4, False)

_RB = 2048
_GRID = NP // _RB


def _tc1_body(x_ref, w_ref, wp_ref, asa_ref, ada_ref, h_ref, hbf_ref,
              aso_ref, ado_ref):
    h = jnp.dot(x_ref[...], w_ref[...], preferred_element_type=jnp.float32)
    h_ref[...] = h
    hp = jnp.dot(x_ref[...], wp_ref[...], preferred_element_type=jnp.float32)
    hbf_ref[...] = hp.astype(jnp.bfloat16)
    rows = (pl.program_id(0) * _RB
            + lax.broadcasted_iota(jnp.int32, (_RB, 1), 0))
    valid = rows < N
    aso = jnp.sum(h * asa_ref[...], axis=1, keepdims=True)
    ado = jnp.sum(h * ada_ref[...], axis=1, keepdims=True)
    aso_ref[...] = jnp.where(valid, aso, NEG)
    ado_ref[...] = jnp.where(valid, ado, NEG)


def _tc2_body(raw_ref, den_ref, h1_ref, as1_ref, ad1_ref, b1_ref, w2_ref,
              as2_ref, ad2_ref, h2_ref, aso_ref, ado_ref):
    al = as1_ref[...] + ad1_ref[...]
    exs = jnp.exp(jnp.maximum(al, 0.2 * al))
    r = raw_ref[0] + raw_ref[1] + exs * h1_ref[...]
    d = den_ref[0] + den_ref[1] + exs
    o1 = r / (d + 1e-16) + b1_ref[...]
    hin = jnp.where(o1 > 0, o1, jnp.exp(jnp.minimum(o1, 0.0)) - 1.0)
    h2 = jnp.dot(hin, w2_ref[...], preferred_element_type=jnp.float32)
    h2_ref[...] = h2
    rows = (pl.program_id(0) * _RB
            + lax.broadcasted_iota(jnp.int32, (_RB, 1), 0))
    valid = rows < N
    aso = jnp.sum(h2 * as2_ref[...], axis=1, keepdims=True)
    ado = jnp.sum(h2 * ad2_ref[...], axis=1, keepdims=True)
    aso_ref[...] = jnp.where(valid, aso, NEG)
    ado_ref[...] = jnp.where(valid, ado, NEG)


def _tc3_body(raw_ref, den_ref, h2_ref, as2_ref, ad2_ref, b2_ref, out_ref):
    al = as2_ref[...] + ad2_ref[...]
    exs = jnp.exp(jnp.maximum(al, 0.2 * al))
    r = raw_ref[0] + raw_ref[1] + exs * h2_ref[...]
    d = den_ref[0] + den_ref[1] + exs
    out_ref[...] = r / (d + 1e-16) + b2_ref[...]


def kernel(x, edge_index, W1, att_src1, att_dst1, b1, W2, att_src2, att_dst2, b2):
    f32 = jnp.float32

    xp = jnp.pad(x, ((0, NP - N), (0, 0)))
    ei3 = jnp.pad(edge_index, ((0, 0), (0, ET - E)),
                  constant_values=N).reshape(2, NW, NCHK, CH)

    zrow64 = jnp.zeros((NP, H), f32)
    zrow16 = jnp.zeros((NP, 16), f32)
    zden = jnp.zeros((NP,), f32)

    w2p = jnp.pad(W2, ((0, 0), (0, 16 - NCLS)))
    as2p = jnp.pad(att_src2, ((0, 0), (0, 16 - NCLS)))
    ad2p = jnp.pad(att_dst2, ((0, 0), (0, 16 - NCLS)))
    b2p = jnp.pad(b2, (0, 16 - NCLS)).reshape(1, 16)

    sigma = np.empty((H,), np.int32)
    for g in range(H // 32):
        for i in range(16):
            sigma[32 * g + 2 * i] = 32 * g + i
            sigma[32 * g + 2 * i + 1] = 32 * g + 16 + i
    w1p = W1[:, sigma]

    h1, h1bf, as1, ad1 = pl.pallas_call(
        _tc1_body,
        grid=(_GRID,),
        in_specs=[
            pl.BlockSpec((_RB, D), lambda i: (i, 0)),
            pl.BlockSpec((D, H), lambda i: (0, 0)),
            pl.BlockSpec((D, H), lambda i: (0, 0)),
            pl.BlockSpec((1, H), lambda i: (0, 0)),
            pl.BlockSpec((1, H), lambda i: (0, 0)),
        ],
        out_specs=[
            pl.BlockSpec((_RB, H), lambda i: (i, 0)),
            pl.BlockSpec((_RB, H), lambda i: (i, 0)),
            pl.BlockSpec((_RB, 1), lambda i: (i, 0)),
            pl.BlockSpec((_RB, 1), lambda i: (i, 0)),
        ],
        out_shape=[
            jax.ShapeDtypeStruct((NP, H), f32),
            jax.ShapeDtypeStruct((NP, H), jnp.bfloat16),
            jax.ShapeDtypeStruct((NP, 1), f32),
            jax.ShapeDtypeStruct((NP, 1), f32),
        ],
    )(xp, W1, w1p, att_src1, att_dst1)

    raw1, den1 = _sc_edge64(ei3, as1.reshape(NP), ad1.reshape(NP),
                            h1bf, zrow64, zden)

    h2, as2, ad2 = pl.pallas_call(
        _tc2_body,
        grid=(_GRID,),
        in_specs=[
            pl.BlockSpec((NC, _RB, H), lambda i: (0, i, 0)),
            pl.BlockSpec((NC, _RB, 1), lambda i: (0, i, 0)),
            pl.BlockSpec((_RB, H), lambda i: (i, 0)),
            pl.BlockSpec((_RB, 1), lambda i: (i, 0)),
            pl.BlockSpec((_RB, 1), lambda i: (i, 0)),
            pl.BlockSpec((1, H), lambda i: (0, 0)),
            pl.BlockSpec((H, 16), lambda i: (0, 0)),
            pl.BlockSpec((1, 16), lambda i: (0, 0)),
            pl.BlockSpec((1, 16), lambda i: (0, 0)),
        ],
        out_specs=[
            pl.BlockSpec((_RB, 16), lambda i: (i, 0)),
            pl.BlockSpec((_RB, 1), lambda i: (i, 0)),
            pl.BlockSpec((_RB, 1), lambda i: (i, 0)),
        ],
        out_shape=[
            jax.ShapeDtypeStruct((NP, 16), f32),
            jax.ShapeDtypeStruct((NP, 1), f32),
            jax.ShapeDtypeStruct((NP, 1), f32),
        ],
    )(raw1, den1.reshape(NC, NP, 1), h1, as1, ad1, b1.reshape(1, H), w2p,
      as2p, ad2p)

    raw2, den2 = _sc_edge16(ei3, as2.reshape(NP), ad2.reshape(NP),
                            h2, zrow16, zden)

    out = pl.pallas_call(
        _tc3_body,
        grid=(_GRID,),
        in_specs=[
            pl.BlockSpec((NC, _RB, 16), lambda i: (0, i, 0)),
            pl.BlockSpec((NC, _RB, 1), lambda i: (0, i, 0)),
            pl.BlockSpec((_RB, 16), lambda i: (i, 0)),
            pl.BlockSpec((_RB, 1), lambda i: (i, 0)),
            pl.BlockSpec((_RB, 1), lambda i: (i, 0)),
            pl.BlockSpec((1, 16), lambda i: (0, 0)),
        ],
        out_specs=pl.BlockSpec((_RB, 16), lambda i: (i, 0)),
        out_shape=jax.ShapeDtypeStruct((NP, 16), f32),
    )(raw2, den2.reshape(NC, NP, 1), h2, as2, ad2, b2p)

    return out[:N, :NCLS]
